# Initial kernel scaffold; baseline (speedup 1.0000x reference)
import jax
import jax.numpy as jnp
from jax.experimental import pallas as pl


def kernel(h, x, mess_graph, W_z, b_z, W_r, U_r, b_ur, W_h, b_h):
    raise NotImplementedError("write your pallas kernel here")


# speedup vs baseline: 3.0925x; 3.0925x over previous
import functools

import jax
import jax.numpy as jnp
from jax import lax
from jax.experimental import pallas as pl
from jax.experimental.pallas import tpu as pltpu
from jax.experimental.pallas import tpu_sc as plsc

M = 320000
K = 8
H = 128
NI = 128
DEPTH = 3

_NC = 2
_NS = 16
NW = _NC * _NS
RPW = M // NW
C = 16
CH = C * K
NCH = RPW // C


@functools.cache
def _make_sc_gather():
    mesh = plsc.VectorSubcoreMesh(core_axis_name="c", subcore_axis_name="s")

    @functools.partial(
        pl.kernel,
        mesh=mesh,
        out_type=jax.ShapeDtypeStruct((M * K, H), jnp.float32),
        scratch_types=[
            pltpu.VMEM((CH,), jnp.int32),
            pltpu.VMEM((CH,), jnp.int32),
            pltpu.VMEM((CH, H), jnp.float32),
            pltpu.VMEM((CH, H), jnp.float32),
            pltpu.SemaphoreType.DMA,
            pltpu.SemaphoreType.DMA,
        ],
    )
    def body(h_hbm, idx_hbm, out_hbm, idx0, idx1, buf0, buf1, sem0, sem1):
        wid = lax.axis_index("s") * _NC + lax.axis_index("c")
        row0 = wid * RPW * K
        idxs = (idx0, idx1)
        bufs = (buf0, buf1)
        sems = (sem0, sem1)

        def fire(c, slot):
            pltpu.sync_copy(idx_hbm.at[pl.ds(row0 + c * CH, CH)], idxs[slot])
            pltpu.async_copy(h_hbm.at[idxs[slot]], bufs[slot], sems[slot])

        fire(0, 0)

        def outer(i, carry):
            for b in range(2):
                c = 2 * i + b

                @pl.when(c < NCH)
                def _():
                    pltpu.make_async_copy(
                        h_hbm.at[idxs[b]], bufs[b], sems[b]).wait()

                    @pl.when(c + 1 < NCH)
                    def _():
                        fire(c + 1, 1 - b)

                    pltpu.sync_copy(
                        bufs[b], out_hbm.at[pl.ds(row0 + c * CH, CH)])
            return carry

        lax.fori_loop(0, (NCH + 1) // 2, outer, 0)

    return body


BMP = 1280


def _pre_body(x_ref, w_ref, b_ref, out_ref):
    out_ref[...] = (
        jnp.dot(x_ref[...], w_ref[...], preferred_element_type=jnp.float32)
        + b_ref[...]
    )


_pre_call = pl.pallas_call(
    _pre_body,
    grid=(M // BMP,),
    in_specs=[
        pl.BlockSpec((BMP, NI), lambda i: (i, 0)),
        pl.BlockSpec((NI, 3 * H), lambda i: (0, 0)),
        pl.BlockSpec((1, 3 * H), lambda i: (0, 0)),
    ],
    out_specs=pl.BlockSpec((BMP, 3 * H), lambda i: (i, 0)),
    out_shape=jax.ShapeDtypeStruct((M, 3 * H), jnp.float32),
)


BM = 512


def _upd_body(nei_ref, pre_ref, wz2_ref, wh2_ref, ur_ref, out_ref):
    nei3 = nei_ref[...]
    nei2 = nei3.reshape(K * BM, H)
    r2 = jnp.dot(nei2, ur_ref[...], preferred_element_type=jnp.float32)
    pre = pre_ref[...]
    r1 = pre[:, 0:H][None, :, :]
    r = 0.5 + 0.5 * jnp.tanh(0.5 * (r1 + r2.reshape(K, BM, H)))
    sum_h = jnp.sum(nei3, axis=0)
    sum_g = jnp.sum(r * nei3, axis=0)
    z = 0.5 + 0.5 * jnp.tanh(0.5 * (
        pre[:, H:2 * H]
        + jnp.dot(sum_h, wz2_ref[...], preferred_element_type=jnp.float32)))
    ph = jnp.tanh(
        pre[:, 2 * H:3 * H]
        + jnp.dot(sum_g, wh2_ref[...], preferred_element_type=jnp.float32))
    hn = (1.0 - z) * sum_h + z * ph
    rid = pl.program_id(0) * BM + lax.broadcasted_iota(jnp.int32, (BM, 1), 0)
    out_ref[...] = jnp.where(rid == 0, 0.0, hn)


_upd_call = pl.pallas_call(
    _upd_body,
    grid=(M // BM,),
    in_specs=[
        pl.BlockSpec((K, BM, H), lambda i: (0, i, 0)),
        pl.BlockSpec((BM, 3 * H), lambda i: (i, 0)),
        pl.BlockSpec((H, H), lambda i: (0, 0)),
        pl.BlockSpec((H, H), lambda i: (0, 0)),
        pl.BlockSpec((H, H), lambda i: (0, 0)),
    ],
    out_specs=pl.BlockSpec((BM, H), lambda i: (i, 0)),
    out_shape=jax.ShapeDtypeStruct((M, H), jnp.float32),
)


def kernel(h, x, mess_graph, W_z, b_z, W_r, U_r, b_ur, W_h, b_h):
    idx = mess_graph.T.reshape(-1).astype(jnp.int32)
    w_cat = jnp.concatenate([W_r, W_z[:NI], W_h[:NI]], axis=1)
    b_cat = jnp.concatenate([b_ur, b_z, b_h]).reshape(1, 3 * H)
    pre = _pre_call(x, w_cat, b_cat)
    wz2 = W_z[NI:]
    wh2 = W_h[NI:]
    sc_gather = _make_sc_gather()
    for _ in range(DEPTH):
        nei = sc_gather(h, idx)
        h = _upd_call(nei.reshape(K, M, H), pre, wz2, wh2, U_r)
    return h


# speedup vs baseline: 3.7306x; 1.2063x over previous
import functools

import jax
import jax.numpy as jnp
from jax import lax
from jax.experimental import pallas as pl
from jax.experimental.pallas import tpu as pltpu
from jax.experimental.pallas import tpu_sc as plsc

M = 320000
K = 8
H = 128
NI = 128
DEPTH = 3

_NC = 2
_NS = 16
NW = _NC * _NS
RPW = M // NW
C = 16
CH = C * K
NCH = RPW // C
GJ = 63
NG = -(-NCH // GJ)
IB = GJ * CH
PAD = NG * IB - RPW * K


@functools.cache
def _make_sc_gather():
    mesh = plsc.VectorSubcoreMesh(core_axis_name="c", subcore_axis_name="s")

    @functools.partial(
        pl.kernel,
        mesh=mesh,
        out_type=jax.ShapeDtypeStruct((M * K, H), jnp.float32),
        scratch_types=[
            pltpu.VMEM((IB,), jnp.int32),
            pltpu.VMEM((IB,), jnp.int32),
            pltpu.VMEM((CH, H), jnp.float32),
            pltpu.VMEM((CH, H), jnp.float32),
            pltpu.VMEM((CH, H), jnp.float32),
            pltpu.SemaphoreType.DMA,
            pltpu.SemaphoreType.DMA,
            pltpu.SemaphoreType.DMA,
            pltpu.SemaphoreType.DMA,
            pltpu.SemaphoreType.DMA,
            pltpu.SemaphoreType.DMA,
            pltpu.SemaphoreType.DMA,
        ],
    )
    def body(h_hbm, idx_hbm, out_hbm, ibuf0, ibuf1, buf0, buf1, buf2,
             gs0, gs1, gs2, ss0, ss1, ss2, isem):
        wid = lax.axis_index("s") * _NC + lax.axis_index("c")
        base = wid * RPW * K
        ibufs = (ibuf0, ibuf1)
        bufs = (buf0, buf1, buf2)
        gsems = (gs0, gs1, gs2)
        ssems = (ss0, ss1, ss2)

        def fire(slot, ibuf, off):
            pltpu.async_copy(
                h_hbm.at[ibuf.at[pl.ds(off, CH)]], bufs[slot], gsems[slot])

        def gwait(slot, ibuf, off):
            pltpu.make_async_copy(
                h_hbm.at[ibuf.at[pl.ds(off, CH)]], bufs[slot],
                gsems[slot]).wait()

        pltpu.sync_copy(idx_hbm.at[pl.ds(base, IB)], ibuf0)
        fire(0, ibuf0, 0)
        fire(1, ibuf0, CH)

        def group(g, carry):
            for par in range(2):

                @pl.when(lax.rem(g, 2) == par)
                def _():
                    cur = ibufs[par]
                    nxt = ibufs[1 - par]
                    for j in range(GJ):
                        c = g * GJ + j

                        @pl.when(c < NCH)
                        def _():
                            if j == 0:
                                @pl.when(g + 1 < NG)
                                def _():
                                    pltpu.async_copy(
                                        idx_hbm.at[
                                            pl.ds(base + (g + 1) * IB, IB)],
                                        nxt, isem)
                            gwait(j % 3, cur, j * CH)
                            if j == GJ - 2:
                                @pl.when(g + 1 < NG)
                                def _():
                                    pltpu.make_async_copy(
                                        idx_hbm.at[
                                            pl.ds(base + (g + 1) * IB, IB)],
                                        nxt, isem).wait()

                            @pl.when(c + 2 < NCH)
                            def _():
                                s2 = (j + 2) % 3
                                @pl.when(c >= 1)
                                def _():
                                    pltpu.make_async_copy(
                                        bufs[s2],
                                        out_hbm.at[
                                            pl.ds(base + (c - 1) * CH, CH)],
                                        ssems[s2]).wait()
                                if j < GJ - 2:
                                    fire(s2, cur, (j + 2) * CH)
                                else:
                                    fire(s2, nxt, (j + 2 - GJ) * CH)

                            pltpu.async_copy(
                                bufs[j % 3],
                                out_hbm.at[pl.ds(base + c * CH, CH)],
                                ssems[j % 3])
            return carry

        lax.fori_loop(0, NG, group, 0)

        for c in (NCH - 3, NCH - 2, NCH - 1):
            pltpu.make_async_copy(
                bufs[c % 3], out_hbm.at[pl.ds(base + c * CH, CH)],
                ssems[c % 3]).wait()

    return body


BMP = 1280


def _pre_body(x_ref, w_ref, b_ref, out_ref):
    out_ref[...] = (
        jnp.dot(x_ref[...], w_ref[...], preferred_element_type=jnp.float32)
        + b_ref[...]
    )


_pre_call = pl.pallas_call(
    _pre_body,
    grid=(M // BMP,),
    in_specs=[
        pl.BlockSpec((BMP, NI), lambda i: (i, 0)),
        pl.BlockSpec((NI, 3 * H), lambda i: (0, 0)),
        pl.BlockSpec((1, 3 * H), lambda i: (0, 0)),
    ],
    out_specs=pl.BlockSpec((BMP, 3 * H), lambda i: (i, 0)),
    out_shape=jax.ShapeDtypeStruct((M, 3 * H), jnp.float32),
)


BM = 512


def _upd_body(nei_ref, pre_ref, wz2_ref, wh2_ref, ur_ref, out_ref):
    nei3 = nei_ref[...]
    nei2 = nei3.reshape(K * BM, H)
    r2 = jnp.dot(nei2, ur_ref[...], preferred_element_type=jnp.float32)
    pre = pre_ref[...]
    r1 = pre[:, 0:H][None, :, :]
    r = 0.5 + 0.5 * jnp.tanh(0.5 * (r1 + r2.reshape(K, BM, H)))
    sum_h = jnp.sum(nei3, axis=0)
    sum_g = jnp.sum(r * nei3, axis=0)
    z = 0.5 + 0.5 * jnp.tanh(0.5 * (
        pre[:, H:2 * H]
        + jnp.dot(sum_h, wz2_ref[...], preferred_element_type=jnp.float32)))
    ph = jnp.tanh(
        pre[:, 2 * H:3 * H]
        + jnp.dot(sum_g, wh2_ref[...], preferred_element_type=jnp.float32))
    hn = (1.0 - z) * sum_h + z * ph
    rid = pl.program_id(0) * BM + lax.broadcasted_iota(jnp.int32, (BM, 1), 0)
    out_ref[...] = jnp.where(rid == 0, 0.0, hn)


_upd_call = pl.pallas_call(
    _upd_body,
    grid=(M // BM,),
    in_specs=[
        pl.BlockSpec((K, BM, H), lambda i: (0, i, 0)),
        pl.BlockSpec((BM, 3 * H), lambda i: (i, 0)),
        pl.BlockSpec((H, H), lambda i: (0, 0)),
        pl.BlockSpec((H, H), lambda i: (0, 0)),
        pl.BlockSpec((H, H), lambda i: (0, 0)),
    ],
    out_specs=pl.BlockSpec((BM, H), lambda i: (i, 0)),
    out_shape=jax.ShapeDtypeStruct((M, H), jnp.float32),
)


def kernel(h, x, mess_graph, W_z, b_z, W_r, U_r, b_ur, W_h, b_h):
    idx = jnp.pad(mess_graph.T.reshape(-1).astype(jnp.int32), (0, PAD))
    w_cat = jnp.concatenate([W_r, W_z[:NI], W_h[:NI]], axis=1)
    b_cat = jnp.concatenate([b_ur, b_z, b_h]).reshape(1, 3 * H)
    pre = _pre_call(x, w_cat, b_cat)
    wz2 = W_z[NI:]
    wh2 = W_h[NI:]
    sc_gather = _make_sc_gather()
    for _ in range(DEPTH):
        nei = sc_gather(h, idx)
        h = _upd_call(nei.reshape(K, M, H), pre, wz2, wh2, U_r)
    return h


# speedup vs baseline: 3.9496x; 1.0587x over previous
import functools

import jax
import jax.numpy as jnp
from jax import lax
from jax.experimental import pallas as pl
from jax.experimental.pallas import tpu as pltpu
from jax.experimental.pallas import tpu_sc as plsc

M = 320000
K = 8
H = 128
NI = 128
DEPTH = 3

_NC = 2
_NS = 16
NW = _NC * _NS
P = 5
MP = M // P
RPW = MP // NW
C = 16
CH = C * K
NCH = RPW // C
GJ = 63
NG = -(-NCH // GJ)
IB = GJ * CH
PAD = NG * IB - RPW * K


@functools.cache
def _make_sc_gather(part):
    mesh = plsc.VectorSubcoreMesh(core_axis_name="c", subcore_axis_name="s")

    @functools.partial(
        pl.kernel,
        mesh=mesh,
        out_type=jax.ShapeDtypeStruct((MP * K, H), jnp.float32),
        scratch_types=[
            pltpu.VMEM((IB,), jnp.int32),
            pltpu.VMEM((IB,), jnp.int32),
            pltpu.VMEM((CH, H), jnp.float32),
            pltpu.VMEM((CH, H), jnp.float32),
            pltpu.VMEM((CH, H), jnp.float32),
            pltpu.SemaphoreType.DMA,
            pltpu.SemaphoreType.DMA,
            pltpu.SemaphoreType.DMA,
            pltpu.SemaphoreType.DMA,
            pltpu.SemaphoreType.DMA,
            pltpu.SemaphoreType.DMA,
            pltpu.SemaphoreType.DMA,
        ],
    )
    def body(h_hbm, idx_hbm, out_hbm, ibuf0, ibuf1, buf0, buf1, buf2,
             gs0, gs1, gs2, ss0, ss1, ss2, isem):
        wid = lax.axis_index("s") * _NC + lax.axis_index("c")
        base = wid * RPW * K
        gbase = part * MP * K + base
        ibufs = (ibuf0, ibuf1)
        bufs = (buf0, buf1, buf2)
        gsems = (gs0, gs1, gs2)
        ssems = (ss0, ss1, ss2)

        def fire(slot, ibuf, off):
            pltpu.async_copy(
                h_hbm.at[ibuf.at[pl.ds(off, CH)]], bufs[slot], gsems[slot])

        def gwait(slot, ibuf, off):
            pltpu.make_async_copy(
                h_hbm.at[ibuf.at[pl.ds(off, CH)]], bufs[slot],
                gsems[slot]).wait()

        pltpu.sync_copy(idx_hbm.at[pl.ds(gbase, IB)], ibuf0)
        fire(0, ibuf0, 0)
        fire(1, ibuf0, CH)

        def group(g, carry):
            for par in range(2):

                @pl.when(lax.rem(g, 2) == par)
                def _():
                    cur = ibufs[par]
                    nxt = ibufs[1 - par]
                    for j in range(GJ):
                        c = g * GJ + j

                        @pl.when(c < NCH)
                        def _():
                            if j == 0:
                                @pl.when(g + 1 < NG)
                                def _():
                                    pltpu.async_copy(
                                        idx_hbm.at[
                                            pl.ds(gbase + (g + 1) * IB, IB)],
                                        nxt, isem)
                            gwait(j % 3, cur, j * CH)
                            if j == GJ - 2:
                                @pl.when(g + 1 < NG)
                                def _():
                                    pltpu.make_async_copy(
                                        idx_hbm.at[
                                            pl.ds(gbase + (g + 1) * IB, IB)],
                                        nxt, isem).wait()

                            @pl.when(c + 2 < NCH)
                            def _():
                                s2 = (j + 2) % 3
                                @pl.when(c >= 1)
                                def _():
                                    pltpu.make_async_copy(
                                        bufs[s2],
                                        out_hbm.at[
                                            pl.ds(base + (c - 1) * CH, CH)],
                                        ssems[s2]).wait()
                                if j < GJ - 2:
                                    fire(s2, cur, (j + 2) * CH)
                                else:
                                    fire(s2, nxt, (j + 2 - GJ) * CH)

                            pltpu.async_copy(
                                bufs[j % 3],
                                out_hbm.at[pl.ds(base + c * CH, CH)],
                                ssems[j % 3])
            return carry

        lax.fori_loop(0, NG, group, 0)

        for c in (NCH - 3, NCH - 2, NCH - 1):
            pltpu.make_async_copy(
                bufs[c % 3], out_hbm.at[pl.ds(base + c * CH, CH)],
                ssems[c % 3]).wait()

    return body


BMP = 1280


def _pre_body(x_ref, w_ref, b_ref, out_ref):
    out_ref[...] = (
        jnp.dot(x_ref[...], w_ref[...], preferred_element_type=jnp.float32)
        + b_ref[...]
    )


_pre_call = pl.pallas_call(
    _pre_body,
    grid=(M // BMP,),
    in_specs=[
        pl.BlockSpec((BMP, NI), lambda i: (i, 0)),
        pl.BlockSpec((NI, 3 * H), lambda i: (0, 0)),
        pl.BlockSpec((1, 3 * H), lambda i: (0, 0)),
    ],
    out_specs=pl.BlockSpec((BMP, 3 * H), lambda i: (i, 0)),
    out_shape=jax.ShapeDtypeStruct((M, 3 * H), jnp.float32),
)


BM = 512


def _upd_body(nei_ref, pre_ref, wz2_ref, wh2_ref, ur_ref, out_ref, *,
              mask_row0):
    nei3 = nei_ref[...]
    nei2 = nei3.reshape(K * BM, H)
    r2 = jnp.dot(nei2, ur_ref[...], preferred_element_type=jnp.float32)
    pre = pre_ref[...]
    r1 = pre[:, 0:H][None, :, :]
    r = 0.5 + 0.5 * jnp.tanh(0.5 * (r1 + r2.reshape(K, BM, H)))
    sum_h = jnp.sum(nei3, axis=0)
    sum_g = jnp.sum(r * nei3, axis=0)
    z = 0.5 + 0.5 * jnp.tanh(0.5 * (
        pre[:, H:2 * H]
        + jnp.dot(sum_h, wz2_ref[...], preferred_element_type=jnp.float32)))
    ph = jnp.tanh(
        pre[:, 2 * H:3 * H]
        + jnp.dot(sum_g, wh2_ref[...], preferred_element_type=jnp.float32))
    hn = (1.0 - z) * sum_h + z * ph
    if mask_row0:
        rid = (pl.program_id(0) * BM
               + lax.broadcasted_iota(jnp.int32, (BM, 1), 0))
        hn = jnp.where(rid == 0, 0.0, hn)
    out_ref[...] = hn


@functools.cache
def _make_upd(part):
    poff = part * (MP // BM)
    return pl.pallas_call(
        functools.partial(_upd_body, mask_row0=(part == 0)),
        grid=(MP // BM,),
        in_specs=[
            pl.BlockSpec((K, BM, H), lambda i: (0, i, 0)),
            pl.BlockSpec((BM, 3 * H), lambda i: (poff + i, 0)),
            pl.BlockSpec((H, H), lambda i: (0, 0)),
            pl.BlockSpec((H, H), lambda i: (0, 0)),
            pl.BlockSpec((H, H), lambda i: (0, 0)),
        ],
        out_specs=pl.BlockSpec((BM, H), lambda i: (i, 0)),
        out_shape=jax.ShapeDtypeStruct((MP, H), jnp.float32),
    )


def kernel(h, x, mess_graph, W_z, b_z, W_r, U_r, b_ur, W_h, b_h):
    idx = mess_graph.astype(jnp.int32).reshape(P, MP, K)
    idx = jnp.pad(idx.transpose(0, 2, 1).reshape(-1), (0, PAD))
    w_cat = jnp.concatenate([W_r, W_z[:NI], W_h[:NI]], axis=1)
    b_cat = jnp.concatenate([b_ur, b_z, b_h]).reshape(1, 3 * H)
    pre = _pre_call(x, w_cat, b_cat)
    wz2 = W_z[NI:]
    wh2 = W_h[NI:]
    for _ in range(DEPTH):
        parts = []
        for p in range(P):
            nei = _make_sc_gather(p)(h, idx)
            parts.append(
                _make_upd(p)(nei.reshape(K, MP, H), pre, wz2, wh2, U_r))
        h = jnp.concatenate(parts, axis=0)
    return h


# speedup vs baseline: 4.0572x; 1.0272x over previous
import functools

import jax
import jax.numpy as jnp
from jax import lax
from jax.experimental import pallas as pl
from jax.experimental.pallas import tpu as pltpu
from jax.experimental.pallas import tpu_sc as plsc

M = 320000
K = 8
H = 128
NI = 128
DEPTH = 3

_NC = 2
_NS = 16
NW = _NC * _NS
P = 5
MP = M // P
RPW = MP // NW
C = 16
CH = C * K
NCH = RPW // C
GJ = 64
NG = -(-NCH // GJ)
IB = GJ * CH
PAD = NG * IB - RPW * K


@functools.cache
def _make_sc_gather(part):
    mesh = plsc.VectorSubcoreMesh(core_axis_name="c", subcore_axis_name="s")

    @functools.partial(
        pl.kernel,
        mesh=mesh,
        out_type=jax.ShapeDtypeStruct((MP * K, H), jnp.float32),
        scratch_types=[
            pltpu.VMEM((IB,), jnp.int32),
            pltpu.VMEM((IB,), jnp.int32),
            pltpu.VMEM((CH, H), jnp.float32),
            pltpu.VMEM((CH, H), jnp.float32),
            pltpu.VMEM((CH, H), jnp.float32),
            pltpu.VMEM((CH, H), jnp.float32),
            pltpu.SemaphoreType.DMA,
            pltpu.SemaphoreType.DMA,
            pltpu.SemaphoreType.DMA,
            pltpu.SemaphoreType.DMA,
            pltpu.SemaphoreType.DMA,
            pltpu.SemaphoreType.DMA,
            pltpu.SemaphoreType.DMA,
            pltpu.SemaphoreType.DMA,
            pltpu.SemaphoreType.DMA,
        ],
    )
    def body(h_hbm, idx_hbm, out_hbm, ibuf0, ibuf1, buf0, buf1, buf2, buf3,
             gs0, gs1, gs2, gs3, ss0, ss1, ss2, ss3, isem):
        wid = lax.axis_index("s") * _NC + lax.axis_index("c")
        base = wid * RPW * K
        gbase = part * MP * K + base
        ibufs = (ibuf0, ibuf1)
        bufs = (buf0, buf1, buf2, buf3)
        gsems = (gs0, gs1, gs2, gs3)
        ssems = (ss0, ss1, ss2, ss3)

        def fire(slot, ibuf, off):
            pltpu.async_copy(
                h_hbm.at[ibuf.at[pl.ds(off, CH)]], bufs[slot], gsems[slot])

        def gwait(slot, ibuf, off):
            pltpu.make_async_copy(
                h_hbm.at[ibuf.at[pl.ds(off, CH)]], bufs[slot],
                gsems[slot]).wait()

        pltpu.sync_copy(idx_hbm.at[pl.ds(gbase, IB)], ibuf0)
        fire(0, ibuf0, 0)
        fire(1, ibuf0, CH)
        fire(2, ibuf0, 2 * CH)

        def group(g, carry):
            for par in range(2):

                @pl.when(lax.rem(g, 2) == par)
                def _():
                    cur = ibufs[par]
                    nxt = ibufs[1 - par]
                    for j in range(GJ):
                        c = g * GJ + j

                        @pl.when(c < NCH)
                        def _():
                            if j == 0:
                                @pl.when(g + 1 < NG)
                                def _():
                                    pltpu.async_copy(
                                        idx_hbm.at[
                                            pl.ds(gbase + (g + 1) * IB, IB)],
                                        nxt, isem)
                            gwait(j % 4, cur, j * CH)
                            if j == GJ - 3:
                                @pl.when(g + 1 < NG)
                                def _():
                                    pltpu.make_async_copy(
                                        idx_hbm.at[
                                            pl.ds(gbase + (g + 1) * IB, IB)],
                                        nxt, isem).wait()

                            @pl.when(c + 3 < NCH)
                            def _():
                                s3 = (j + 3) % 4
                                @pl.when(c >= 1)
                                def _():
                                    pltpu.make_async_copy(
                                        bufs[s3],
                                        out_hbm.at[
                                            pl.ds(base + (c - 1) * CH, CH)],
                                        ssems[s3]).wait()
                                if j < GJ - 3:
                                    fire(s3, cur, (j + 3) * CH)
                                else:
                                    fire(s3, nxt, (j + 3 - GJ) * CH)

                            pltpu.async_copy(
                                bufs[j % 4],
                                out_hbm.at[pl.ds(base + c * CH, CH)],
                                ssems[j % 4])
            return carry

        lax.fori_loop(0, NG, group, 0)

        for c in (NCH - 4, NCH - 3, NCH - 2, NCH - 1):
            pltpu.make_async_copy(
                bufs[c % 4], out_hbm.at[pl.ds(base + c * CH, CH)],
                ssems[c % 4]).wait()

    return body


BMP = 1280


def _pre_body(x_ref, w_ref, b_ref, out_ref):
    out_ref[...] = (
        jnp.dot(x_ref[...], w_ref[...], preferred_element_type=jnp.float32)
        + b_ref[...]
    )


_pre_call = pl.pallas_call(
    _pre_body,
    grid=(M // BMP,),
    in_specs=[
        pl.BlockSpec((BMP, NI), lambda i: (i, 0)),
        pl.BlockSpec((NI, 3 * H), lambda i: (0, 0)),
        pl.BlockSpec((1, 3 * H), lambda i: (0, 0)),
    ],
    out_specs=pl.BlockSpec((BMP, 3 * H), lambda i: (i, 0)),
    out_shape=jax.ShapeDtypeStruct((M, 3 * H), jnp.float32),
)


BM = 512


def _upd_body(nei_ref, pre_ref, wz2_ref, wh2_ref, ur_ref, hbuf_ref, out_ref,
              *, mask_row0):
    del hbuf_ref
    nei3 = nei_ref[...]
    nei2 = nei3.reshape(K * BM, H)
    r2 = jnp.dot(nei2, ur_ref[...], preferred_element_type=jnp.float32)
    pre = pre_ref[...]
    r1 = pre[:, 0:H][None, :, :]
    r = 0.5 + 0.5 * jnp.tanh(0.5 * (r1 + r2.reshape(K, BM, H)))
    sum_h = jnp.sum(nei3, axis=0)
    sum_g = jnp.sum(r * nei3, axis=0)
    z = 0.5 + 0.5 * jnp.tanh(0.5 * (
        pre[:, H:2 * H]
        + jnp.dot(sum_h, wz2_ref[...], preferred_element_type=jnp.float32)))
    ph = jnp.tanh(
        pre[:, 2 * H:3 * H]
        + jnp.dot(sum_g, wh2_ref[...], preferred_element_type=jnp.float32))
    hn = (1.0 - z) * sum_h + z * ph
    if mask_row0:
        rid = (pl.program_id(0) * BM
               + lax.broadcasted_iota(jnp.int32, (BM, 1), 0))
        hn = jnp.where(rid == 0, 0.0, hn)
    out_ref[...] = hn


@functools.cache
def _make_upd(part):
    poff = part * (MP // BM)
    return pl.pallas_call(
        functools.partial(_upd_body, mask_row0=(part == 0)),
        grid=(MP // BM,),
        in_specs=[
            pl.BlockSpec((K, BM, H), lambda i: (0, i, 0)),
            pl.BlockSpec((BM, 3 * H), lambda i: (poff + i, 0)),
            pl.BlockSpec((H, H), lambda i: (0, 0)),
            pl.BlockSpec((H, H), lambda i: (0, 0)),
            pl.BlockSpec((H, H), lambda i: (0, 0)),
            pl.BlockSpec(memory_space=pl.ANY),
        ],
        out_specs=pl.BlockSpec((BM, H), lambda i: (poff + i, 0)),
        out_shape=jax.ShapeDtypeStruct((M, H), jnp.float32),
        input_output_aliases={5: 0},
    )


def kernel(h, x, mess_graph, W_z, b_z, W_r, U_r, b_ur, W_h, b_h):
    idx = mess_graph.astype(jnp.int32).reshape(P, MP, K)
    idx = jnp.pad(idx.transpose(0, 2, 1).reshape(-1), (0, PAD))
    w_cat = jnp.concatenate([W_r, W_z[:NI], W_h[:NI]], axis=1)
    b_cat = jnp.concatenate([b_ur, b_z, b_h]).reshape(1, 3 * H)
    pre = _pre_call(x, w_cat, b_cat)
    wz2 = W_z[NI:]
    wh2 = W_h[NI:]
    spares = [jnp.zeros((M, H), jnp.float32), jnp.zeros((M, H), jnp.float32)]
    for t in range(DEPTH):
        hb = spares[0] if t == 0 else (spares[1] if t == 1 else None)
        if t == 2:
            hb = prev_tab
        neis = [_make_sc_gather(p)(h, idx) for p in range(P)]
        for p in range(P):
            hb = _make_upd(p)(neis[p].reshape(K, MP, H), pre, wz2, wh2, U_r,
                              hb)
        prev_tab = h
        h = hb
    return h


# speedup vs baseline: 4.6789x; 1.1532x over previous
import functools

import jax
import jax.numpy as jnp
from jax import lax
from jax.experimental import pallas as pl
from jax.experimental.pallas import tpu as pltpu
from jax.experimental.pallas import tpu_sc as plsc

M = 320000
K = 8
H = 128
NI = 128
DEPTH = 3

_NC = 2
_NS = 16
NW = _NC * _NS
P = 5
MP = M // P
RPW = MP // NW
C = 16
CH = C * K
NCH = RPW // C
GJ = 64
NG = -(-NCH // GJ)
IB = GJ * CH
PAD = NG * IB - RPW * K


@functools.cache
def _make_sc_gather(part):
    mesh = plsc.VectorSubcoreMesh(core_axis_name="c", subcore_axis_name="s")

    @functools.partial(
        pl.kernel,
        mesh=mesh,
        out_type=jax.ShapeDtypeStruct((MP * K, H), jnp.float32),
        scratch_types=[
            pltpu.VMEM((IB,), jnp.int32),
            pltpu.VMEM((IB,), jnp.int32),
            pltpu.VMEM((CH, H), jnp.float32),
            pltpu.VMEM((CH, H), jnp.float32),
            pltpu.VMEM((CH, H), jnp.float32),
            pltpu.VMEM((CH, H), jnp.float32),
            pltpu.SemaphoreType.DMA,
            pltpu.SemaphoreType.DMA,
            pltpu.SemaphoreType.DMA,
            pltpu.SemaphoreType.DMA,
            pltpu.SemaphoreType.DMA,
            pltpu.SemaphoreType.DMA,
            pltpu.SemaphoreType.DMA,
            pltpu.SemaphoreType.DMA,
            pltpu.SemaphoreType.DMA,
        ],
    )
    def body(h_hbm, idx_hbm, out_hbm, ibuf0, ibuf1, buf0, buf1, buf2, buf3,
             gs0, gs1, gs2, gs3, ss0, ss1, ss2, ss3, isem):
        wid = lax.axis_index("s") * _NC + lax.axis_index("c")
        base = wid * RPW * K
        gbase = part * MP * K + base
        ibufs = (ibuf0, ibuf1)
        bufs = (buf0, buf1, buf2, buf3)
        gsems = (gs0, gs1, gs2, gs3)
        ssems = (ss0, ss1, ss2, ss3)

        def fire(slot, ibuf, off):
            pltpu.async_copy(
                h_hbm.at[ibuf.at[pl.ds(off, CH)]], bufs[slot], gsems[slot])

        def gwait(slot, ibuf, off):
            pltpu.make_async_copy(
                h_hbm.at[ibuf.at[pl.ds(off, CH)]], bufs[slot],
                gsems[slot]).wait()

        pltpu.sync_copy(idx_hbm.at[pl.ds(gbase, IB)], ibuf0)
        fire(0, ibuf0, 0)
        fire(1, ibuf0, CH)
        fire(2, ibuf0, 2 * CH)

        def group(g, carry):
            for par in range(2):

                @pl.when(lax.rem(g, 2) == par)
                def _():
                    cur = ibufs[par]
                    nxt = ibufs[1 - par]
                    for j in range(GJ):
                        c = g * GJ + j

                        @pl.when(c < NCH)
                        def _():
                            if j == 0:
                                @pl.when(g + 1 < NG)
                                def _():
                                    pltpu.async_copy(
                                        idx_hbm.at[
                                            pl.ds(gbase + (g + 1) * IB, IB)],
                                        nxt, isem)
                            gwait(j % 4, cur, j * CH)
                            if j == GJ - 3:
                                @pl.when(g + 1 < NG)
                                def _():
                                    pltpu.make_async_copy(
                                        idx_hbm.at[
                                            pl.ds(gbase + (g + 1) * IB, IB)],
                                        nxt, isem).wait()

                            @pl.when(c + 3 < NCH)
                            def _():
                                s3 = (j + 3) % 4
                                @pl.when(c >= 1)
                                def _():
                                    pltpu.make_async_copy(
                                        bufs[s3],
                                        out_hbm.at[
                                            pl.ds(base + (c - 1) * CH, CH)],
                                        ssems[s3]).wait()
                                if j < GJ - 3:
                                    fire(s3, cur, (j + 3) * CH)
                                else:
                                    fire(s3, nxt, (j + 3 - GJ) * CH)

                            pltpu.async_copy(
                                bufs[j % 4],
                                out_hbm.at[pl.ds(base + c * CH, CH)],
                                ssems[j % 4])
            return carry

        lax.fori_loop(0, NG, group, 0)

        for c in (NCH - 4, NCH - 3, NCH - 2, NCH - 1):
            pltpu.make_async_copy(
                bufs[c % 4], out_hbm.at[pl.ds(base + c * CH, CH)],
                ssems[c % 4]).wait()

    return body


BMP = 1280


def _pre_body(x_ref, w_ref, b_ref, out_ref):
    out_ref[...] = (
        jnp.dot(x_ref[...], w_ref[...], preferred_element_type=jnp.float32)
        + b_ref[...]
    ).astype(jnp.bfloat16)


_pre_call = pl.pallas_call(
    _pre_body,
    grid=(M // BMP,),
    in_specs=[
        pl.BlockSpec((BMP, NI), lambda i: (i, 0)),
        pl.BlockSpec((NI, 3 * H), lambda i: (0, 0)),
        pl.BlockSpec((1, 3 * H), lambda i: (0, 0)),
    ],
    out_specs=pl.BlockSpec((BMP, 3 * H), lambda i: (i, 0)),
    out_shape=jax.ShapeDtypeStruct((M, 3 * H), jnp.bfloat16),
)


BM = 800


def _upd_body(nei_ref, pre_ref, wz2_ref, wh2_ref, ur_ref, *rest, mask_row0):
    out_ref = rest[-1]
    nei3 = nei_ref[...]
    nei2 = nei3.reshape(K * BM, H)
    r2 = jnp.dot(nei2, ur_ref[...], preferred_element_type=jnp.float32)
    pre = pre_ref[...].astype(jnp.float32)
    r1 = pre[:, 0:H][None, :, :]
    r = 0.5 + 0.5 * jnp.tanh(0.5 * (r1 + r2.reshape(K, BM, H)))
    sum_h = jnp.sum(nei3, axis=0)
    sum_g = jnp.sum(r * nei3, axis=0)
    z = 0.5 + 0.5 * jnp.tanh(0.5 * (
        pre[:, H:2 * H]
        + jnp.dot(sum_h, wz2_ref[...], preferred_element_type=jnp.float32)))
    ph = jnp.tanh(
        pre[:, 2 * H:3 * H]
        + jnp.dot(sum_g, wh2_ref[...], preferred_element_type=jnp.float32))
    hn = (1.0 - z) * sum_h + z * ph
    if mask_row0:
        rid = (pl.program_id(0) * BM
               + lax.broadcasted_iota(jnp.int32, (BM, 1), 0))
        hn = jnp.where(rid == 0, 0.0, hn)
    out_ref[...] = hn


@functools.cache
def _make_upd(part):
    poff = part * (MP // BM)
    in_specs = [
        pl.BlockSpec((K, BM, H), lambda i: (0, i, 0)),
        pl.BlockSpec((BM, 3 * H), lambda i: (poff + i, 0)),
        pl.BlockSpec((H, H), lambda i: (0, 0)),
        pl.BlockSpec((H, H), lambda i: (0, 0)),
        pl.BlockSpec((H, H), lambda i: (0, 0)),
    ]
    aliases = {}
    if part > 0:
        in_specs += [pl.BlockSpec(memory_space=pl.ANY)]
        aliases = {5: 0}
    return pl.pallas_call(
        functools.partial(_upd_body, mask_row0=(part == 0)),
        grid=(MP // BM,),
        in_specs=in_specs,
        out_specs=pl.BlockSpec((BM, H), lambda i: (poff + i, 0)),
        out_shape=jax.ShapeDtypeStruct((M, H), jnp.float32),
        input_output_aliases=aliases,
    )


def kernel(h, x, mess_graph, W_z, b_z, W_r, U_r, b_ur, W_h, b_h):
    idx = mess_graph.astype(jnp.int32).reshape(P, MP, K)
    idx = jnp.pad(idx.transpose(0, 2, 1).reshape(-1), (0, PAD))
    w_cat = jnp.concatenate([W_r, W_z[:NI], W_h[:NI]], axis=1)
    b_cat = jnp.concatenate([b_ur, b_z, b_h]).reshape(1, 3 * H)
    pre = _pre_call(x, w_cat, b_cat)
    wz2 = W_z[NI:]
    wh2 = W_h[NI:]
    for _ in range(DEPTH):
        neis = [_make_sc_gather(p)(h, idx) for p in range(P)]
        hf = _make_upd(0)(neis[0].reshape(K, MP, H), pre, wz2, wh2, U_r)
        for p in range(1, P):
            hf = _make_upd(p)(neis[p].reshape(K, MP, H), pre, wz2, wh2, U_r,
                              hf)
        h = hf
    return h
